# baseline (device time: 369718 ns/iter reference)
import jax
import jax.numpy as jnp
from jax import lax
from jax.experimental import pallas as pl
from jax.experimental.pallas import tpu as pltpu

N_DEV = 32
NB = 2
NBLK = 2 * NB
S = 4


def _gelu(y):
    c = 0.7978845608028654
    return 0.5 * y * (1.0 + jnp.tanh(c * (y + 0.044715 * y * y * y)))


def kernel(x, w_mat):
    m_total, k_per = x.shape
    _, n = w_mat.shape
    m_per = m_total // N_DEV
    width = n // NBLK

    def body(x_ref, w_ref, out_ref, buf, part, send_sems, recv_sems,
             credit_r, credit_l):
        my = lax.axis_index("i")
        left = jnp.mod(my - 1, N_DEV)
        right = jnp.mod(my + 1, N_DEV)

        barrier_sem = pltpu.get_barrier_semaphore()
        for nbr in (left, right):
            pl.semaphore_signal(
                barrier_sem, inc=1,
                device_id=(nbr,), device_id_type=pl.DeviceIdType.MESH,
            )
        pl.semaphore_wait(barrier_sem, 2)

        part[:, :] = jnp.dot(x_ref[:, :], w_ref[:, :],
                             preferred_element_type=jnp.float32)

        def partial(c, idx):
            return part[pl.ds(c * m_per, m_per), pl.ds(idx * width, width)]

        def send_chunk(idx, s):
            if idx < NB:
                return jnp.mod(my - 1 - s, N_DEV)
            return jnp.mod(my + 1 + s, N_DEV)

        def recv_chunk(idx, s):
            if idx < NB:
                return jnp.mod(my - 2 - s, N_DEV)
            return jnp.mod(my + 2 + s, N_DEV)

        def desc(idx, s):
            to = right if idx < NB else left
            return pltpu.make_async_remote_copy(
                src_ref=buf.at[s % S, idx],
                dst_ref=buf.at[(s + 1) % S, idx],
                send_sem=send_sems.at[s % S, idx],
                recv_sem=recv_sems.at[(s + 1) % S, idx],
                device_id=(to,),
                device_id_type=pl.DeviceIdType.MESH,
            )

        def credit_sem(idx):
            return credit_r if idx < NB else credit_l

        def upstream(idx):
            return left if idx < NB else right

        for idx in range(NBLK):
            buf[0, idx] = partial(send_chunk(idx, 0), idx)

        for s in range(N_DEV - 1):
            for idx in range(NBLK):
                if s >= 1:
                    desc(idx, s - 1).wait()
                    if s <= N_DEV - S:
                        pl.semaphore_signal(
                            credit_sem(idx), inc=1,
                            device_id=(upstream(idx),),
                            device_id_type=pl.DeviceIdType.MESH,
                        )
                    buf[s % S, idx] = buf[s % S, idx] + partial(
                        recv_chunk(idx, s - 1), idx)
                    if s >= S - 1:
                        pl.semaphore_wait(credit_sem(idx), 1)
                desc(idx, s).start()

        for idx in range(NBLK):
            desc(idx, N_DEV - 2).wait()
            y = buf[(N_DEV - 1) % S, idx] + partial(my, idx)
            out_ref[:, pl.ds(idx * width, width)] = _gelu(y)

    return pl.pallas_call(
        body,
        out_shape=jax.ShapeDtypeStruct((m_per, n), jnp.float32),
        in_specs=[
            pl.BlockSpec(memory_space=pltpu.VMEM),
            pl.BlockSpec(memory_space=pltpu.VMEM),
        ],
        out_specs=pl.BlockSpec(memory_space=pltpu.VMEM),
        scratch_shapes=[
            pltpu.VMEM((S, NBLK, m_per, width), jnp.float32),
            pltpu.VMEM((m_total, n), jnp.float32),
            pltpu.SemaphoreType.DMA((S, NBLK)),
            pltpu.SemaphoreType.DMA((S, NBLK)),
            pltpu.SemaphoreType.REGULAR,
            pltpu.SemaphoreType.REGULAR,
        ],
        compiler_params=pltpu.CompilerParams(
            collective_id=0,
            vmem_limit_bytes=100 * 1024 * 1024,
        ),
    )(x, w_mat)


# device time: 198748 ns/iter; 1.8602x vs baseline; 1.8602x over previous
import jax
import jax.numpy as jnp
from jax import lax
from jax.experimental import pallas as pl
from jax.experimental.pallas import tpu as pltpu

N_DEV = 32
NB = 2
NBLK = 2 * NB
S = 4

RING_DEV = [
    0, 1, 9, 8, 16, 17, 25, 24,
    27, 26, 18, 19, 11, 10, 13, 12,
    20, 21, 29, 28, 31, 30, 22, 23,
    15, 14, 6, 7, 4, 5, 2, 3,
]
assert sorted(RING_DEV) == list(range(N_DEV))


def _gelu(y):
    c = 0.7978845608028654
    return 0.5 * y * (1.0 + jnp.tanh(c * (y + 0.044715 * y * y * y)))


def kernel(x, w_mat):
    m_total, k_per = x.shape
    _, n = w_mat.shape
    m_per = m_total // N_DEV
    width = n // NBLK

    ring = jnp.asarray(RING_DEV, dtype=jnp.int32)
    pos = jnp.zeros((N_DEV,), jnp.int32).at[ring].set(
        jnp.arange(N_DEV, dtype=jnp.int32))
    my = lax.axis_index("i")
    q = pos[my]
    steps = jnp.arange(N_DEV, dtype=jnp.int32)
    chunks_f = ring[jnp.mod(q - 1 - steps, N_DEV)]
    chunks_b = ring[jnp.mod(q + 1 + steps, N_DEV)]
    nbrs = jnp.stack([ring[jnp.mod(q + 1, N_DEV)],
                      ring[jnp.mod(q - 1, N_DEV)]])

    def body(x_ref, w_ref, chunks_f_ref, chunks_b_ref, nbrs_ref,
             out_ref, buf, part, send_sems, recv_sems,
             credit_f, credit_b):
        rnext = nbrs_ref[0]
        rprev = nbrs_ref[1]

        barrier_sem = pltpu.get_barrier_semaphore()
        for nbr in (rnext, rprev):
            pl.semaphore_signal(
                barrier_sem, inc=1,
                device_id=(nbr,), device_id_type=pl.DeviceIdType.MESH,
            )
        pl.semaphore_wait(barrier_sem, 2)

        part[:, :] = jnp.dot(x_ref[:, :], w_ref[:, :],
                             preferred_element_type=jnp.float32)

        def partial(c, idx):
            return part[pl.ds(c * m_per, m_per), pl.ds(idx * width, width)]

        def chunk_at(idx, j):
            return chunks_f_ref[j] if idx < NB else chunks_b_ref[j]

        def desc(idx, s):
            to = rnext if idx < NB else rprev
            return pltpu.make_async_remote_copy(
                src_ref=buf.at[s % S, idx],
                dst_ref=buf.at[(s + 1) % S, idx],
                send_sem=send_sems.at[s % S, idx],
                recv_sem=recv_sems.at[(s + 1) % S, idx],
                device_id=(to,),
                device_id_type=pl.DeviceIdType.MESH,
            )

        def credit_sem(idx):
            return credit_f if idx < NB else credit_b

        def upstream(idx):
            return rprev if idx < NB else rnext

        for idx in range(NBLK):
            buf[0, idx] = partial(chunk_at(idx, 0), idx)

        for s in range(N_DEV - 1):
            for idx in range(NBLK):
                if s >= 1:
                    desc(idx, s - 1).wait()
                    if s <= N_DEV - S:
                        pl.semaphore_signal(
                            credit_sem(idx), inc=1,
                            device_id=(upstream(idx),),
                            device_id_type=pl.DeviceIdType.MESH,
                        )
                    buf[s % S, idx] = buf[s % S, idx] + partial(
                        chunk_at(idx, s), idx)
                    if s >= S - 1:
                        pl.semaphore_wait(credit_sem(idx), 1)
                desc(idx, s).start()

        for idx in range(NBLK):
            desc(idx, N_DEV - 2).wait()
            y = buf[(N_DEV - 1) % S, idx] + partial(chunk_at(idx, N_DEV - 1), idx)
            out_ref[:, pl.ds(idx * width, width)] = _gelu(y)

    return pl.pallas_call(
        body,
        out_shape=jax.ShapeDtypeStruct((m_per, n), jnp.float32),
        in_specs=[
            pl.BlockSpec(memory_space=pltpu.VMEM),
            pl.BlockSpec(memory_space=pltpu.VMEM),
            pl.BlockSpec(memory_space=pltpu.SMEM),
            pl.BlockSpec(memory_space=pltpu.SMEM),
            pl.BlockSpec(memory_space=pltpu.SMEM),
        ],
        out_specs=pl.BlockSpec(memory_space=pltpu.VMEM),
        scratch_shapes=[
            pltpu.VMEM((S, NBLK, m_per, width), jnp.float32),
            pltpu.VMEM((m_total, n), jnp.float32),
            pltpu.SemaphoreType.DMA((S, NBLK)),
            pltpu.SemaphoreType.DMA((S, NBLK)),
            pltpu.SemaphoreType.REGULAR,
            pltpu.SemaphoreType.REGULAR,
        ],
        compiler_params=pltpu.CompilerParams(
            collective_id=0,
            vmem_limit_bytes=100 * 1024 * 1024,
        ),
    )(x, w_mat, chunks_f, chunks_b, nbrs)


# device time: 192690 ns/iter; 1.9187x vs baseline; 1.0314x over previous
import jax
import jax.numpy as jnp
from jax import lax
from jax.experimental import pallas as pl
from jax.experimental.pallas import tpu as pltpu

N_DEV = 32
NB = 2
NBLK = 2 * NB
S = 4

RING_DEV = [
    0, 1, 9, 8, 16, 17, 25, 24,
    27, 26, 18, 19, 11, 10, 13, 12,
    20, 21, 29, 28, 31, 30, 22, 23,
    15, 14, 6, 7, 4, 5, 2, 3,
]
assert sorted(RING_DEV) == list(range(N_DEV))


def _gelu(y):
    c = 0.7978845608028654
    return 0.5 * y * (1.0 + jnp.tanh(c * (y + 0.044715 * y * y * y)))


def kernel(x, w_mat):
    m_total, k_per = x.shape
    _, n = w_mat.shape
    m_per = m_total // N_DEV
    width = n // NBLK

    ring = jnp.asarray(RING_DEV, dtype=jnp.int32)
    pos = jnp.zeros((N_DEV,), jnp.int32).at[ring].set(
        jnp.arange(N_DEV, dtype=jnp.int32))
    my = lax.axis_index("i")
    q = pos[my]
    steps = jnp.arange(N_DEV, dtype=jnp.int32)
    chunks_f = ring[jnp.mod(q - 1 - steps, N_DEV)]
    chunks_b = ring[jnp.mod(q + 1 + steps, N_DEV)]
    nbrs = jnp.stack([ring[jnp.mod(q + 1, N_DEV)],
                      ring[jnp.mod(q - 1, N_DEV)]])

    def body(x_ref, w_ref, chunks_f_ref, chunks_b_ref, nbrs_ref,
             out_ref, buf, spart, send_sems, recv_sems,
             credit_f, credit_b):
        rnext = nbrs_ref[0]
        rprev = nbrs_ref[1]

        barrier_sem = pltpu.get_barrier_semaphore()
        for nbr in (rnext, rprev):
            pl.semaphore_signal(
                barrier_sem, inc=1,
                device_id=(nbr,), device_id_type=pl.DeviceIdType.MESH,
            )
        pl.semaphore_wait(barrier_sem, 2)

        def sched_dot(j):
            c = chunks_f_ref[j]
            spart[j] = jnp.dot(x_ref[pl.ds(c * m_per, m_per), :],
                               w_ref[:, :],
                               preferred_element_type=jnp.float32)

        def pslice(idx, s):
            if idx < NB:
                j = s
            else:
                j = N_DEV - 2 - s if s <= N_DEV - 2 else N_DEV - 1
            return spart[j, :, pl.ds(idx * width, width)]

        def desc(idx, s):
            to = rnext if idx < NB else rprev
            return pltpu.make_async_remote_copy(
                src_ref=buf.at[s % S, idx],
                dst_ref=buf.at[(s + 1) % S, idx],
                send_sem=send_sems.at[s % S, idx],
                recv_sem=recv_sems.at[(s + 1) % S, idx],
                device_id=(to,),
                device_id_type=pl.DeviceIdType.MESH,
            )

        def credit_sem(idx):
            return credit_f if idx < NB else credit_b

        def upstream(idx):
            return rprev if idx < NB else rnext

        sched_dot(0)
        sched_dot(N_DEV - 2)
        for idx in range(NBLK):
            buf[0, idx] = pslice(idx, 0)
        for j in (1, N_DEV - 3, N_DEV - 1):
            sched_dot(j)

        for s in range(N_DEV - 1):
            for idx in range(NBLK):
                if s >= 1:
                    desc(idx, s - 1).wait()
                    if s <= N_DEV - S:
                        pl.semaphore_signal(
                            credit_sem(idx), inc=1,
                            device_id=(upstream(idx),),
                            device_id_type=pl.DeviceIdType.MESH,
                        )
                    buf[s % S, idx] = buf[s % S, idx] + pslice(idx, s)
                    if s >= S - 1:
                        pl.semaphore_wait(credit_sem(idx), 1)
                desc(idx, s).start()
            if s <= 13:
                sched_dot(s + 2)
                if 28 - s != s + 2:
                    sched_dot(28 - s)

        for idx in range(NBLK):
            desc(idx, N_DEV - 2).wait()
            y = buf[(N_DEV - 1) % S, idx] + pslice(idx, N_DEV - 1)
            out_ref[:, pl.ds(idx * width, width)] = _gelu(y)

    return pl.pallas_call(
        body,
        out_shape=jax.ShapeDtypeStruct((m_per, n), jnp.float32),
        in_specs=[
            pl.BlockSpec(memory_space=pltpu.VMEM),
            pl.BlockSpec(memory_space=pltpu.VMEM),
            pl.BlockSpec(memory_space=pltpu.SMEM),
            pl.BlockSpec(memory_space=pltpu.SMEM),
            pl.BlockSpec(memory_space=pltpu.SMEM),
        ],
        out_specs=pl.BlockSpec(memory_space=pltpu.VMEM),
        scratch_shapes=[
            pltpu.VMEM((S, NBLK, m_per, width), jnp.float32),
            pltpu.VMEM((N_DEV, m_per, n), jnp.float32),
            pltpu.SemaphoreType.DMA((S, NBLK)),
            pltpu.SemaphoreType.DMA((S, NBLK)),
            pltpu.SemaphoreType.REGULAR,
            pltpu.SemaphoreType.REGULAR,
        ],
        compiler_params=pltpu.CompilerParams(
            collective_id=0,
            vmem_limit_bytes=100 * 1024 * 1024,
        ),
    )(x, w_mat, chunks_f, chunks_b, nbrs)
